# baseline (speedup 1.0000x reference)
import functools

import jax
import jax.numpy as jnp
from jax import lax
from jax.experimental import pallas as pl
from jax.experimental.pallas import tpu as pltpu
from jax.experimental.pallas import tpu_sc as plsc

_N, _C = 32, 768
_K = _C // 2
_ONE_BITS = 0x3F800000
_B = 56
_CHUNKS = _C // 16


def _mlp_body(mask_ref, W1_ref, b1_ref, W2_ref, b2_ref, m_ref):
    h = jnp.dot(mask_ref[:], W1_ref[:], preferred_element_type=jnp.float32)
    h = jnp.maximum(h + b1_ref[:], 0.0)
    z = jnp.dot(h, W2_ref[:], preferred_element_type=jnp.float32) + b2_ref[:]
    m_ref[:] = jax.nn.sigmoid(z)


def _make_sc_mask():
    mesh = plsc.VectorSubcoreMesh(core_axis_name="c", subcore_axis_name="s")

    @functools.partial(
        pl.kernel,
        out_type=jax.ShapeDtypeStruct((_N, _C), jnp.float32),
        mesh=mesh,
        scratch_types=[
            pltpu.VMEM((_C,), jnp.float32),
            pltpu.VMEM((_C,), jnp.int32),
            pltpu.VMEM((_C,), jnp.float32),
        ],
    )
    def sc_mask(m_hbm, cm_hbm, row_v, bits_v, out_v):
        wid = lax.axis_index("s") * 2 + lax.axis_index("c")
        pltpu.sync_copy(m_hbm.at[wid], row_v)
        for j in range(_CHUNKS):
            sl = pl.ds(j * 16, 16)
            bits_v[sl] = lax.bitcast_convert_type(row_v[sl], jnp.int32)

        ones16 = jnp.full((16,), 1, jnp.int32)
        zeros16 = jnp.full((16,), 0, jnp.int32)

        def bs_step(_, carry):
            lo, hi = carry
            mid = (lo + hi) >> 1
            acc = zeros16
            for j in range(_CHUNKS):
                le = bits_v[pl.ds(j * 16, 16)] <= mid
                acc = acc + jnp.where(le, ones16, zeros16)
            cnt = acc[0]
            for i in range(1, 16):
                cnt = cnt + acc[i]
            ge = cnt >= _K
            return jnp.where(ge, lo, mid), jnp.where(ge, mid, hi)

        _, kth = lax.fori_loop(
            0, 31, bs_step, (jnp.int32(-1), jnp.int32(_ONE_BITS)))
        ones16f = jnp.full((16,), 1.0, jnp.float32)
        zeros16f = jnp.full((16,), 0.0, jnp.float32)
        for j in range(_CHUNKS):
            sl = pl.ds(j * 16, 16)
            out_v[sl] = jnp.where(bits_v[sl] > kth, ones16f, zeros16f)
        pltpu.sync_copy(out_v, cm_hbm.at[wid])

    return sc_mask


def _swap_body(cm_ref, lst_ref, gui_ref, ol_ref, og_ref):
    sel = (cm_ref[:] > 0.5)[None, :, :]
    l = lst_ref[:]
    g = gui_ref[:]
    ol_ref[:] = jnp.where(sel, g, l)
    og_ref[:] = jnp.where(sel, l, g)


def kernel(lst, gui, mask, W1, b1, W2, b2):
    N, C, H, W = lst.shape
    HW = H * W

    m = pl.pallas_call(
        _mlp_body,
        out_shape=jax.ShapeDtypeStruct((N, C), jnp.float32),
    )(mask, W1, b1.reshape(1, C), W2, b2.reshape(1, C))

    cm = _make_sc_mask()(m)

    lst2 = lst.transpose(2, 3, 0, 1).reshape(HW, N, C)
    gui2 = gui.transpose(2, 3, 0, 1).reshape(HW, N, C)

    const = lambda i: (0, 0)
    blk = lambda i: (i, 0, 0)
    ol, og = pl.pallas_call(
        _swap_body,
        grid=(HW // _B,),
        in_specs=[
            pl.BlockSpec((N, C), const),
            pl.BlockSpec((_B, N, C), blk),
            pl.BlockSpec((_B, N, C), blk),
        ],
        out_specs=(
            pl.BlockSpec((_B, N, C), blk),
            pl.BlockSpec((_B, N, C), blk),
        ),
        out_shape=(
            jax.ShapeDtypeStruct((HW, N, C), jnp.float32),
            jax.ShapeDtypeStruct((HW, N, C), jnp.float32),
        ),
        compiler_params=pltpu.CompilerParams(
            dimension_semantics=("arbitrary",),
        ),
    )(cm, lst2, gui2)

    ol = ol.reshape(H, W, N, C).transpose(2, 3, 0, 1)
    og = og.reshape(H, W, N, C).transpose(2, 3, 0, 1)
    return ol, og, m


# speedup vs baseline: 1.1813x; 1.1813x over previous
import jax
import jax.numpy as jnp
from jax import lax
from jax.experimental import pallas as pl
from jax.experimental.pallas import tpu as pltpu

_N, _C = 32, 768
_K = _C // 2
_ONE_BITS = 0x3F800000
_B = 56


def _body(mask_ref, W1_ref, b1_ref, W2_ref, b2_ref, lst_ref, gui_ref,
          m_ref, ol_ref, og_ref, cm_ref):
    @pl.when(pl.program_id(0) == 0)
    def _():
        h = jnp.dot(mask_ref[:], W1_ref[:], preferred_element_type=jnp.float32)
        h = jnp.maximum(h + b1_ref[:], 0.0)
        z = jnp.dot(h, W2_ref[:], preferred_element_type=jnp.float32) + b2_ref[:]
        m = jax.nn.sigmoid(z)
        m_ref[:] = m

        bits = lax.bitcast_convert_type(m, jnp.int32)

        def step(_, carry):
            lo, hi = carry
            mid = (lo + hi) >> 1
            cnt = jnp.sum((bits <= mid).astype(jnp.int32), axis=1, keepdims=True)
            ge = cnt >= _K
            return jnp.where(ge, lo, mid), jnp.where(ge, mid, hi)

        lo0 = jnp.full((_N, 1), -1, jnp.int32)
        hi0 = jnp.full((_N, 1), _ONE_BITS, jnp.int32)
        _, kth_bits = lax.fori_loop(0, 31, step, (lo0, hi0))
        cm_ref[:] = (bits > kth_bits).astype(jnp.float32)

    sel = (cm_ref[:] > 0.5)[None, :, :]
    l = lst_ref[:]
    g = gui_ref[:]
    ol_ref[:] = jnp.where(sel, g, l)
    og_ref[:] = jnp.where(sel, l, g)


def kernel(lst, gui, mask, W1, b1, W2, b2):
    N, C, H, W = lst.shape
    HW = H * W

    lst2 = lst.transpose(2, 3, 0, 1).reshape(HW, N, C)
    gui2 = gui.transpose(2, 3, 0, 1).reshape(HW, N, C)

    const = lambda i: (0, 0)
    blk = lambda i: (i, 0, 0)
    m, ol, og = pl.pallas_call(
        _body,
        grid=(HW // _B,),
        in_specs=[
            pl.BlockSpec(mask.shape, const),
            pl.BlockSpec(W1.shape, const),
            pl.BlockSpec((1, C), const),
            pl.BlockSpec(W2.shape, const),
            pl.BlockSpec((1, C), const),
            pl.BlockSpec((_B, N, C), blk),
            pl.BlockSpec((_B, N, C), blk),
        ],
        out_specs=(
            pl.BlockSpec((N, C), const),
            pl.BlockSpec((_B, N, C), blk),
            pl.BlockSpec((_B, N, C), blk),
        ),
        out_shape=(
            jax.ShapeDtypeStruct((N, C), jnp.float32),
            jax.ShapeDtypeStruct((HW, N, C), jnp.float32),
            jax.ShapeDtypeStruct((HW, N, C), jnp.float32),
        ),
        scratch_shapes=[pltpu.VMEM((N, C), jnp.float32)],
        compiler_params=pltpu.CompilerParams(
            dimension_semantics=("arbitrary",),
        ),
    )(mask, W1, b1.reshape(1, C), W2, b2.reshape(1, C), lst2, gui2)

    ol = ol.reshape(H, W, N, C).transpose(2, 3, 0, 1)
    og = og.reshape(H, W, N, C).transpose(2, 3, 0, 1)
    return ol, og, m


# speedup vs baseline: 1.2089x; 1.0233x over previous
import jax
import jax.numpy as jnp
from jax import lax
from jax.experimental import pallas as pl
from jax.experimental.pallas import tpu as pltpu

_N, _C = 32, 768
_K = _C // 2
_ONE_BITS = 0x3F800000
_B = 64


def _body(mask_ref, W1_ref, b1_ref, W2_ref, b2_ref, lst_ref, gui_ref,
          m_ref, ol_ref, og_ref, cm_ref):
    @pl.when(pl.program_id(0) == 0)
    def _():
        h = jnp.dot(mask_ref[:], W1_ref[:], preferred_element_type=jnp.float32)
        h = jnp.maximum(h + b1_ref[:], 0.0)
        z = jnp.dot(h, W2_ref[:], preferred_element_type=jnp.float32) + b2_ref[:]
        m = jax.nn.sigmoid(z)
        m_ref[:] = m

        bits = lax.bitcast_convert_type(m, jnp.int32)

        def step(_, carry):
            lo, hi = carry
            mid = (lo + hi) >> 1
            cnt = jnp.sum((bits <= mid).astype(jnp.int32), axis=1, keepdims=True)
            ge = cnt >= _K
            return jnp.where(ge, lo, mid), jnp.where(ge, mid, hi)

        lo0 = jnp.full((_N, 1), -1, jnp.int32)
        hi0 = jnp.full((_N, 1), _ONE_BITS, jnp.int32)
        _, kth_bits = lax.fori_loop(0, 31, step, (lo0, hi0))
        cm_ref[:] = (bits > kth_bits).astype(jnp.float32)

    sel = (cm_ref[:] > 0.5)[None, :, :]
    l = lst_ref[:]
    g = gui_ref[:]
    ol_ref[:] = jnp.where(sel, g, l)
    og_ref[:] = jnp.where(sel, l, g)


def kernel(lst, gui, mask, W1, b1, W2, b2):
    N, C, H, W = lst.shape
    HW = H * W

    lst2 = lst.transpose(2, 3, 0, 1).reshape(HW, N, C)
    gui2 = gui.transpose(2, 3, 0, 1).reshape(HW, N, C)

    const = lambda i: (0, 0)
    blk = lambda i: (i, 0, 0)
    m, ol, og = pl.pallas_call(
        _body,
        grid=(-(-HW // _B),),
        in_specs=[
            pl.BlockSpec(mask.shape, const),
            pl.BlockSpec(W1.shape, const),
            pl.BlockSpec((1, C), const),
            pl.BlockSpec(W2.shape, const),
            pl.BlockSpec((1, C), const),
            pl.BlockSpec((_B, N, C), blk),
            pl.BlockSpec((_B, N, C), blk),
        ],
        out_specs=(
            pl.BlockSpec((N, C), const),
            pl.BlockSpec((_B, N, C), blk),
            pl.BlockSpec((_B, N, C), blk),
        ),
        out_shape=(
            jax.ShapeDtypeStruct((N, C), jnp.float32),
            jax.ShapeDtypeStruct((HW, N, C), jnp.float32),
            jax.ShapeDtypeStruct((HW, N, C), jnp.float32),
        ),
        scratch_shapes=[pltpu.VMEM((N, C), jnp.float32)],
        compiler_params=pltpu.CompilerParams(
            dimension_semantics=("arbitrary",),
        ),
    )(mask, W1, b1.reshape(1, C), W2, b2.reshape(1, C), lst2, gui2)

    ol = ol.reshape(H, W, N, C).transpose(2, 3, 0, 1)
    og = og.reshape(H, W, N, C).transpose(2, 3, 0, 1)
    return ol, og, m


# speedup vs baseline: 1.2123x; 1.0028x over previous
import jax
import jax.numpy as jnp
from jax import lax
from jax.experimental import pallas as pl
from jax.experimental.pallas import tpu as pltpu

_N, _C = 32, 768
_K = _C // 2
_ONE_BITS = 0x3F800000
_B = 68


def _body(mask_ref, W1_ref, b1_ref, W2_ref, b2_ref, lst_ref, gui_ref,
          m_ref, ol_ref, og_ref, cm_ref):
    @pl.when(pl.program_id(0) == 0)
    def _():
        h = jnp.dot(mask_ref[:], W1_ref[:], preferred_element_type=jnp.float32)
        h = jnp.maximum(h + b1_ref[:], 0.0)
        z = jnp.dot(h, W2_ref[:], preferred_element_type=jnp.float32) + b2_ref[:]
        m = jax.nn.sigmoid(z)
        m_ref[:] = m

        bits = lax.bitcast_convert_type(m, jnp.int32)

        def step(_, carry):
            lo, hi = carry
            mid = (lo + hi) >> 1
            cnt = jnp.sum((bits <= mid).astype(jnp.int32), axis=1, keepdims=True)
            ge = cnt >= _K
            return jnp.where(ge, lo, mid), jnp.where(ge, mid, hi)

        lo0 = jnp.full((_N, 1), -1, jnp.int32)
        hi0 = jnp.full((_N, 1), _ONE_BITS, jnp.int32)
        _, kth_bits = lax.fori_loop(0, 31, step, (lo0, hi0))
        cm_ref[:] = (bits > kth_bits).astype(jnp.float32)

    sel = (cm_ref[:] > 0.5)[None, :, :]
    l = lst_ref[:]
    g = gui_ref[:]
    ol_ref[:] = jnp.where(sel, g, l)
    og_ref[:] = jnp.where(sel, l, g)


def kernel(lst, gui, mask, W1, b1, W2, b2):
    N, C, H, W = lst.shape
    HW = H * W

    lst2 = lst.transpose(2, 3, 0, 1).reshape(HW, N, C)
    gui2 = gui.transpose(2, 3, 0, 1).reshape(HW, N, C)

    const = lambda i: (0, 0)
    blk = lambda i: (i, 0, 0)
    m, ol, og = pl.pallas_call(
        _body,
        grid=(-(-HW // _B),),
        in_specs=[
            pl.BlockSpec(mask.shape, const),
            pl.BlockSpec(W1.shape, const),
            pl.BlockSpec((1, C), const),
            pl.BlockSpec(W2.shape, const),
            pl.BlockSpec((1, C), const),
            pl.BlockSpec((_B, N, C), blk),
            pl.BlockSpec((_B, N, C), blk),
        ],
        out_specs=(
            pl.BlockSpec((N, C), const),
            pl.BlockSpec((_B, N, C), blk),
            pl.BlockSpec((_B, N, C), blk),
        ),
        out_shape=(
            jax.ShapeDtypeStruct((N, C), jnp.float32),
            jax.ShapeDtypeStruct((HW, N, C), jnp.float32),
            jax.ShapeDtypeStruct((HW, N, C), jnp.float32),
        ),
        scratch_shapes=[pltpu.VMEM((N, C), jnp.float32)],
        compiler_params=pltpu.CompilerParams(
            dimension_semantics=("arbitrary",),
        ),
    )(mask, W1, b1.reshape(1, C), W2, b2.reshape(1, C), lst2, gui2)

    ol = ol.reshape(H, W, N, C).transpose(2, 3, 0, 1)
    og = og.reshape(H, W, N, C).transpose(2, 3, 0, 1)
    return ol, og, m
